# baseline (speedup 1.0000x reference)
import functools
import jax
import jax.numpy as jnp
from jax import lax
from jax.experimental import pallas as pl
from jax.experimental.pallas import tpu as pltpu
from jax.experimental.pallas import tpu_sc as plsc

N_NODES = 10000
N_EDGES = 320000
ROW_BLK = 2000

NSC = 2
NSUB = 16
NW = NSC * NSUB
EB = 128
NT = 80
E_PAD = NW * NT * EB
NBLK_TOT = E_PAD // EB
NACC = 10240
STRIPE = NACC // NSUB

_mesh = plsc.VectorSubcoreMesh(core_axis_name="c", subcore_axis_name="s")


def _scat_body(g_hbm, src_hbm, dst_hbm, zero_hbm, out_hbm,
               idx_s, idx_d, r0, acc, sem):
    c = lax.axis_index("c")
    s = lax.axis_index("s")
    w = c * NSUB + s
    pltpu.sync_copy(zero_hbm, acc.at[pl.ds(s * STRIPE, STRIPE)])
    plsc.subcore_barrier()

    def body(t, carry):
        blk = w * NT + t
        pltpu.sync_copy(src_hbm.at[blk], idx_s)
        pltpu.async_copy(g_hbm.at[idx_s], r0, sem).wait()
        pltpu.sync_copy(dst_hbm.at[blk], idx_d)
        pltpu.sync_copy(r0, acc.at[idx_d], add=True)
        return carry

    lax.fori_loop(0, NT, body, 0)

    plsc.subcore_barrier()
    row0 = c * NACC + s * STRIPE
    pltpu.sync_copy(acc.at[pl.ds(s * STRIPE, STRIPE)],
                    out_hbm.at[pl.ds(row0, STRIPE)])


_scat_kernel = pl.kernel(
    _scat_body,
    out_type=jax.ShapeDtypeStruct((NSC * NACC, 128), jnp.float32),
    mesh=_mesh,
    scratch_types=[
        pltpu.VMEM((EB,), jnp.int32),
        pltpu.VMEM((EB,), jnp.int32),
        pltpu.VMEM((EB, 128), jnp.float32),
        pltpu.VMEM_SHARED((NACC, 128), jnp.float32),
        pltpu.SemaphoreType.DMA,
    ],
)


def _sc_scatter(g, src2d, dst2d, zero128):
    f = g.shape[1]
    parts = []
    for j in range(f // 128):
        gj = g[:, j * 128:(j + 1) * 128]
        p = _scat_kernel(gj, src2d, dst2d, zero128)
        parts.append((p[:N_NODES], p[NACC:NACC + N_NODES]))
    return parts


def _dinv_body(p0_ref, p1_ref, out_ref):
    out_ref[...] = lax.rsqrt(p0_ref[...] + p1_ref[...] + 1.0)


def _dinv(degp):
    p0 = degp[:N_NODES, :1]
    p1 = degp[NACC:NACC + N_NODES, :1]
    return pl.pallas_call(
        _dinv_body,
        grid=(N_NODES // ROW_BLK,),
        in_specs=[
            pl.BlockSpec((ROW_BLK, 1), lambda i: (i, 0)),
            pl.BlockSpec((ROW_BLK, 1), lambda i: (i, 0)),
        ],
        out_specs=pl.BlockSpec((ROW_BLK, 1), lambda i: (i, 0)),
        out_shape=jax.ShapeDtypeStruct((N_NODES, 1), jnp.float32),
    )(p0, p1)


def _mm_scale_body(x_ref, w_ref, dinv_ref, out_ref):
    h = jnp.dot(x_ref[...], w_ref[...], preferred_element_type=jnp.float32)
    out_ref[...] = h * dinv_ref[...]


def _mm_scale(x, w, dinv):
    n, fin = x.shape
    fout = w.shape[1]
    return pl.pallas_call(
        _mm_scale_body,
        grid=(n // ROW_BLK,),
        in_specs=[
            pl.BlockSpec((ROW_BLK, fin), lambda i: (i, 0)),
            pl.BlockSpec((fin, fout), lambda i: (0, 0)),
            pl.BlockSpec((ROW_BLK, 1), lambda i: (i, 0)),
        ],
        out_specs=pl.BlockSpec((ROW_BLK, fout), lambda i: (i, 0)),
        out_shape=jax.ShapeDtypeStruct((n, fout), jnp.float32),
    )(x, w, dinv)


def _relu_combine(part_refs, g_ref, dinv_ref, b_ref):
    nchunk = len(part_refs) // 2
    cols = []
    for j in range(nchunk):
        p0 = part_refs[2 * j][...]
        p1 = part_refs[2 * j + 1][...]
        gj = g_ref[:, j * 128:(j + 1) * 128]
        bj = b_ref[:, j * 128:(j + 1) * 128]
        cols.append((p0 + p1 + gj) * dinv_ref[...] + bj)
    x = cols[0] if nchunk == 1 else jnp.concatenate(cols, axis=1)
    return jnp.maximum(x, 0.0)


def _combine_mm(parts, g, dinv, b, w):
    n, fin = g.shape
    fout = w.shape[1]
    nchunk = len(parts)

    def body(*refs):
        part_refs = refs[:2 * nchunk]
        g_ref, dinv_ref, b_ref, w_ref, out_ref = refs[2 * nchunk:]
        x = _relu_combine(part_refs, g_ref, dinv_ref, b_ref)
        h = jnp.dot(x, w_ref[...], preferred_element_type=jnp.float32)
        out_ref[...] = h * dinv_ref[...]

    chunk_spec = pl.BlockSpec((ROW_BLK, 128), lambda i: (i, 0))
    in_specs = [chunk_spec] * (2 * nchunk) + [
        pl.BlockSpec((ROW_BLK, fin), lambda i: (i, 0)),
        pl.BlockSpec((ROW_BLK, 1), lambda i: (i, 0)),
        pl.BlockSpec((1, fin), lambda i: (0, 0)),
        pl.BlockSpec((fin, fout), lambda i: (0, 0)),
    ]
    flat = [p for pair in parts for p in pair]
    return pl.pallas_call(
        body,
        grid=(n // ROW_BLK,),
        in_specs=in_specs,
        out_specs=pl.BlockSpec((ROW_BLK, fout), lambda i: (i, 0)),
        out_shape=jax.ShapeDtypeStruct((n, fout), jnp.float32),
    )(*flat, g, dinv, b.reshape(1, fin), w)


def _combine_final(parts, g, dinv, b):
    n, f = g.shape
    nchunk = len(parts)

    def body(*refs):
        part_refs = refs[:2 * nchunk]
        g_ref, dinv_ref, b_ref, out_ref = refs[2 * nchunk:]
        out_ref[...] = _relu_combine(part_refs, g_ref, dinv_ref, b_ref)

    chunk_spec = pl.BlockSpec((ROW_BLK, 128), lambda i: (i, 0))
    in_specs = [chunk_spec] * (2 * nchunk) + [
        pl.BlockSpec((ROW_BLK, f), lambda i: (i, 0)),
        pl.BlockSpec((ROW_BLK, 1), lambda i: (i, 0)),
        pl.BlockSpec((1, f), lambda i: (0, 0)),
    ]
    flat = [p for pair in parts for p in pair]
    return pl.pallas_call(
        body,
        grid=(n // ROW_BLK,),
        in_specs=in_specs,
        out_specs=pl.BlockSpec((ROW_BLK, f), lambda i: (i, 0)),
        out_shape=jax.ShapeDtypeStruct((n, f), jnp.float32),
    )(*flat, g, dinv, b.reshape(1, f))


def kernel(v, edge_index, W1, b1, W2, b2, W3, b3):
    pad = E_PAD - N_EDGES
    src2d = jnp.concatenate(
        [edge_index[0], jnp.zeros((pad,), jnp.int32)]).reshape(NBLK_TOT, EB)
    dst2d = jnp.concatenate(
        [edge_index[1], jnp.full((pad,), N_NODES, jnp.int32)]).reshape(NBLK_TOT, EB)
    zero128 = jnp.zeros((STRIPE, 128), jnp.float32)

    ones_tab = jnp.ones((N_NODES, 128), jnp.float32)
    src2d_deg = jnp.zeros((NBLK_TOT, EB), jnp.int32)
    degp = _scat_kernel(ones_tab, src2d_deg, dst2d, zero128)
    dinv = _dinv(degp)

    g1 = _mm_scale(v, W1, dinv)
    g2 = _combine_mm(_sc_scatter(g1, src2d, dst2d, zero128), g1, dinv, b1, W2)
    g3 = _combine_mm(_sc_scatter(g2, src2d, dst2d, zero128), g2, dinv, b2, W3)
    return _combine_final(_sc_scatter(g3, src2d, dst2d, zero128), g3, dinv, b3)


# speedup vs baseline: 10.0498x; 10.0498x over previous
import functools
import jax
import jax.numpy as jnp
from jax import lax
from jax.experimental import pallas as pl
from jax.experimental.pallas import tpu as pltpu
from jax.experimental.pallas import tpu_sc as plsc

N_NODES = 10000
N_EDGES = 320000
ROW_BLK = 2000

NSC = 2
NSUB = 16
NW = NSC * NSUB
EB = 80
NT = 125
E_PAD = NW * NT * EB
NBLK_TOT = E_PAD // EB
NACC = 10240
STRIPE = NACC // NSUB

_mesh = plsc.VectorSubcoreMesh(core_axis_name="c", subcore_axis_name="s")


def _scat_body(g_hbm, src_hbm, dst_hbm, zero_hbm, out_hbm, *rest):
    idxS = rest[0:4]
    idxD = rest[4:8]
    rows = rest[8:12]
    acc = rest[12]
    sg = rest[13:17]
    c = lax.axis_index("c")
    s = lax.axis_index("s")
    w = c * NSUB + s
    pltpu.sync_copy(zero_hbm, acc.at[pl.ds(s * STRIPE, STRIPE)])
    plsc.subcore_barrier()

    def fetch(t, b):
        pltpu.sync_copy(src_hbm.at[w * NT + t], idxS[b])
        pltpu.async_copy(g_hbm.at[idxS[b]], rows[b], sg[b])

    def g_wait(b):
        pltpu.make_async_copy(g_hbm.at[idxS[b]], rows[b], sg[b]).wait()

    def consume(t, b):
        g_wait(b)
        pltpu.sync_copy(dst_hbm.at[w * NT + t], idxD[b])
        pltpu.sync_copy(rows[b], acc.at[idxD[b]], add=True)

    for t in range(4):
        fetch(t, t)

    def body(i, carry):
        for bp in range(4):
            t = 4 * i + bp
            consume(t, bp)
            tn = jnp.minimum(t + 4, NT - 1)
            pl.when(t + 4 < NT)(lambda: fetch(tn, bp))
        return carry

    lax.fori_loop(0, (NT - 1) // 4, body, 0)
    consume(NT - 1, 0)

    plsc.subcore_barrier()
    row0 = c * NACC + s * STRIPE
    pltpu.sync_copy(acc.at[pl.ds(s * STRIPE, STRIPE)],
                    out_hbm.at[pl.ds(row0, STRIPE)])


_scat_kernel = pl.kernel(
    _scat_body,
    out_type=jax.ShapeDtypeStruct((NSC * NACC, 128), jnp.float32),
    mesh=_mesh,
    scratch_types=(
        [pltpu.VMEM((EB,), jnp.int32)] * 8
        + [pltpu.VMEM((EB, 128), jnp.float32)] * 4
        + [pltpu.VMEM_SHARED((NACC, 128), jnp.float32)]
        + [pltpu.SemaphoreType.DMA] * 4
    ),
)


def _deg_body(dst_hbm, ones_hbm, zero_hbm, out_hbm, idx_d, ones_v, acc):
    c = lax.axis_index("c")
    s = lax.axis_index("s")
    w = c * NSUB + s
    pltpu.sync_copy(ones_hbm, ones_v)
    pltpu.sync_copy(zero_hbm, acc.at[pl.ds(s * STRIPE, STRIPE)])
    plsc.subcore_barrier()

    def body(t, carry):
        pltpu.sync_copy(dst_hbm.at[w * NT + t], idx_d)
        pltpu.sync_copy(ones_v, acc.at[idx_d], add=True)
        return carry

    lax.fori_loop(0, NT, body, 0)
    plsc.subcore_barrier()
    row0 = c * NACC + s * STRIPE
    pltpu.sync_copy(acc.at[pl.ds(s * STRIPE, STRIPE)],
                    out_hbm.at[pl.ds(row0, STRIPE)])


_deg_kernel = pl.kernel(
    _deg_body,
    out_type=jax.ShapeDtypeStruct((NSC * NACC, 128), jnp.float32),
    mesh=_mesh,
    scratch_types=[
        pltpu.VMEM((EB,), jnp.int32),
        pltpu.VMEM((EB, 128), jnp.float32),
        pltpu.VMEM_SHARED((NACC, 128), jnp.float32),
    ],
)


def _sc_scatter(g, src2d, dst2d, zero128):
    f = g.shape[1]
    parts = []
    for j in range(f // 128):
        gj = g[:, j * 128:(j + 1) * 128]
        p = _scat_kernel(gj, src2d, dst2d, zero128)
        parts.append((p[:N_NODES], p[NACC:NACC + N_NODES]))
    return parts


def _dinv_body(p0_ref, p1_ref, out_ref):
    out_ref[...] = lax.rsqrt(p0_ref[...] + p1_ref[...] + 1.0)


def _dinv(degp):
    p0 = degp[:N_NODES, :1]
    p1 = degp[NACC:NACC + N_NODES, :1]
    return pl.pallas_call(
        _dinv_body,
        grid=(N_NODES // ROW_BLK,),
        in_specs=[
            pl.BlockSpec((ROW_BLK, 1), lambda i: (i, 0)),
            pl.BlockSpec((ROW_BLK, 1), lambda i: (i, 0)),
        ],
        out_specs=pl.BlockSpec((ROW_BLK, 1), lambda i: (i, 0)),
        out_shape=jax.ShapeDtypeStruct((N_NODES, 1), jnp.float32),
    )(p0, p1)


def _mm_scale_body(x_ref, w_ref, dinv_ref, out_ref):
    h = jnp.dot(x_ref[...], w_ref[...], preferred_element_type=jnp.float32)
    out_ref[...] = h * dinv_ref[...]


def _mm_scale(x, w, dinv):
    n, fin = x.shape
    fout = w.shape[1]
    return pl.pallas_call(
        _mm_scale_body,
        grid=(n // ROW_BLK,),
        in_specs=[
            pl.BlockSpec((ROW_BLK, fin), lambda i: (i, 0)),
            pl.BlockSpec((fin, fout), lambda i: (0, 0)),
            pl.BlockSpec((ROW_BLK, 1), lambda i: (i, 0)),
        ],
        out_specs=pl.BlockSpec((ROW_BLK, fout), lambda i: (i, 0)),
        out_shape=jax.ShapeDtypeStruct((n, fout), jnp.float32),
    )(x, w, dinv)


def _relu_combine(part_refs, g_ref, dinv_ref, b_ref):
    nchunk = len(part_refs) // 2
    cols = []
    for j in range(nchunk):
        p0 = part_refs[2 * j][...]
        p1 = part_refs[2 * j + 1][...]
        gj = g_ref[:, j * 128:(j + 1) * 128]
        bj = b_ref[:, j * 128:(j + 1) * 128]
        cols.append((p0 + p1 + gj) * dinv_ref[...] + bj)
    x = cols[0] if nchunk == 1 else jnp.concatenate(cols, axis=1)
    return jnp.maximum(x, 0.0)


def _combine_mm(parts, g, dinv, b, w):
    n, fin = g.shape
    fout = w.shape[1]
    nchunk = len(parts)

    def body(*refs):
        part_refs = refs[:2 * nchunk]
        g_ref, dinv_ref, b_ref, w_ref, out_ref = refs[2 * nchunk:]
        x = _relu_combine(part_refs, g_ref, dinv_ref, b_ref)
        h = jnp.dot(x, w_ref[...], preferred_element_type=jnp.float32)
        out_ref[...] = h * dinv_ref[...]

    chunk_spec = pl.BlockSpec((ROW_BLK, 128), lambda i: (i, 0))
    in_specs = [chunk_spec] * (2 * nchunk) + [
        pl.BlockSpec((ROW_BLK, fin), lambda i: (i, 0)),
        pl.BlockSpec((ROW_BLK, 1), lambda i: (i, 0)),
        pl.BlockSpec((1, fin), lambda i: (0, 0)),
        pl.BlockSpec((fin, fout), lambda i: (0, 0)),
    ]
    flat = [p for pair in parts for p in pair]
    return pl.pallas_call(
        body,
        grid=(n // ROW_BLK,),
        in_specs=in_specs,
        out_specs=pl.BlockSpec((ROW_BLK, fout), lambda i: (i, 0)),
        out_shape=jax.ShapeDtypeStruct((n, fout), jnp.float32),
    )(*flat, g, dinv, b.reshape(1, fin), w)


def _combine_final(parts, g, dinv, b):
    n, f = g.shape
    nchunk = len(parts)

    def body(*refs):
        part_refs = refs[:2 * nchunk]
        g_ref, dinv_ref, b_ref, out_ref = refs[2 * nchunk:]
        out_ref[...] = _relu_combine(part_refs, g_ref, dinv_ref, b_ref)

    chunk_spec = pl.BlockSpec((ROW_BLK, 128), lambda i: (i, 0))
    in_specs = [chunk_spec] * (2 * nchunk) + [
        pl.BlockSpec((ROW_BLK, f), lambda i: (i, 0)),
        pl.BlockSpec((ROW_BLK, 1), lambda i: (i, 0)),
        pl.BlockSpec((1, f), lambda i: (0, 0)),
    ]
    flat = [p for pair in parts for p in pair]
    return pl.pallas_call(
        body,
        grid=(n // ROW_BLK,),
        in_specs=in_specs,
        out_specs=pl.BlockSpec((ROW_BLK, f), lambda i: (i, 0)),
        out_shape=jax.ShapeDtypeStruct((n, f), jnp.float32),
    )(*flat, g, dinv, b.reshape(1, f))


def kernel(v, edge_index, W1, b1, W2, b2, W3, b3):
    src2d = edge_index[0].reshape(NBLK_TOT, EB)
    dst2d = edge_index[1].reshape(NBLK_TOT, EB)
    zero128 = jnp.zeros((STRIPE, 128), jnp.float32)
    ones128 = jnp.ones((EB, 128), jnp.float32)

    degp = _deg_kernel(dst2d, ones128, zero128)
    dinv = _dinv(degp)

    g1 = _mm_scale(v, W1, dinv)
    g2 = _combine_mm(_sc_scatter(g1, src2d, dst2d, zero128), g1, dinv, b1, W2)
    g3 = _combine_mm(_sc_scatter(g2, src2d, dst2d, zero128), g2, dinv, b2, W3)
    return _combine_final(_sc_scatter(g3, src2d, dst2d, zero128), g3, dinv, b3)


# speedup vs baseline: 15.2725x; 1.5197x over previous
import functools
import jax
import jax.numpy as jnp
from jax import lax
from jax.experimental import pallas as pl
from jax.experimental.pallas import tpu as pltpu
from jax.experimental.pallas import tpu_sc as plsc

N_NODES = 10000
N_EDGES = 320000
ROW_BLK = 2000

NSC = 2
NSUB = 16
NW = NSC * NSUB
EB = 80
NT = 125
E_PAD = NW * NT * EB
NBLK_TOT = E_PAD // EB
NACC = 10240
STRIPE = NACC // NSUB

_mesh = plsc.VectorSubcoreMesh(core_axis_name="c", subcore_axis_name="s")


def _scat_body(g_hbm, edges_hbm, zero_hbm, out_hbm, *rest):
    idx2 = rest[0:4]
    rows = rest[4:8]
    acc = rest[8]
    si = rest[9:13]
    sg = rest[13:17]
    c = lax.axis_index("c")
    s = lax.axis_index("s")
    w = c * NSUB + s
    pltpu.sync_copy(zero_hbm, acc.at[pl.ds(s * STRIPE, STRIPE)])
    plsc.subcore_barrier()

    def fetch_idx(t, b):
        pltpu.async_copy(edges_hbm.at[w * NT + t], idx2[b], si[b])

    def issue_gather(t, b):
        pltpu.make_async_copy(edges_hbm.at[w * NT + t], idx2[b], si[b]).wait()
        pltpu.async_copy(g_hbm.at[idx2[b].at[0]], rows[b], sg[b])

    def consume(t, b):
        pltpu.make_async_copy(g_hbm.at[idx2[b].at[0]], rows[b], sg[b]).wait()
        pltpu.sync_copy(rows[b], acc.at[idx2[b].at[1]], add=True)

    for t in range(4):
        fetch_idx(t, t)
    for t in range(2):
        issue_gather(t, t)

    def body(i, carry):
        for bp in range(4):
            t = 4 * i + bp
            consume(t, bp)
            t4 = jnp.minimum(t + 4, NT - 1)
            pl.when(t + 4 < NT)(lambda: fetch_idx(t4, bp))
            t2 = jnp.minimum(t + 2, NT - 1)
            b2 = (bp + 2) % 4
            pl.when(t + 2 < NT)(lambda: issue_gather(t2, b2))
        return carry

    lax.fori_loop(0, (NT - 1) // 4, body, 0)
    consume(NT - 1, 0)

    plsc.subcore_barrier()
    row0 = c * NACC + s * STRIPE
    pltpu.sync_copy(acc.at[pl.ds(s * STRIPE, STRIPE)],
                    out_hbm.at[pl.ds(row0, STRIPE)])


_scat_kernel = pl.kernel(
    _scat_body,
    out_type=jax.ShapeDtypeStruct((NSC * NACC, 128), jnp.float32),
    mesh=_mesh,
    scratch_types=(
        [pltpu.VMEM((2, EB), jnp.int32)] * 4
        + [pltpu.VMEM((EB, 128), jnp.float32)] * 4
        + [pltpu.VMEM_SHARED((NACC, 128), jnp.float32)]
        + [pltpu.SemaphoreType.DMA] * 8
    ),
)


def _deg_body(dst_hbm, ones_hbm, zero_hbm, out_hbm, *rest):
    idxD = rest[0:4]
    ones_v = rest[4]
    acc = rest[5]
    si = rest[6:10]
    c = lax.axis_index("c")
    s = lax.axis_index("s")
    w = c * NSUB + s
    pltpu.sync_copy(ones_hbm, ones_v)
    pltpu.sync_copy(zero_hbm, acc.at[pl.ds(s * STRIPE, STRIPE)])
    plsc.subcore_barrier()

    def fetch_idx(t, b):
        pltpu.async_copy(dst_hbm.at[w * NT + t], idxD[b], si[b])

    def consume(t, b):
        pltpu.make_async_copy(dst_hbm.at[w * NT + t], idxD[b], si[b]).wait()
        pltpu.sync_copy(ones_v, acc.at[idxD[b]], add=True)

    for t in range(4):
        fetch_idx(t, t)

    def body(i, carry):
        for bp in range(4):
            t = 4 * i + bp
            consume(t, bp)
            t4 = jnp.minimum(t + 4, NT - 1)
            pl.when(t + 4 < NT)(lambda: fetch_idx(t4, bp))
        return carry

    lax.fori_loop(0, (NT - 1) // 4, body, 0)
    consume(NT - 1, 0)
    plsc.subcore_barrier()
    row0 = c * NACC + s * STRIPE
    pltpu.sync_copy(acc.at[pl.ds(s * STRIPE, STRIPE)],
                    out_hbm.at[pl.ds(row0, STRIPE)])


_deg_kernel = pl.kernel(
    _deg_body,
    out_type=jax.ShapeDtypeStruct((NSC * NACC, 128), jnp.float32),
    mesh=_mesh,
    scratch_types=(
        [pltpu.VMEM((EB,), jnp.int32)] * 4
        + [pltpu.VMEM((EB, 128), jnp.float32)]
        + [pltpu.VMEM_SHARED((NACC, 128), jnp.float32)]
        + [pltpu.SemaphoreType.DMA] * 4
    ),
)


def _sc_scatter(g, edges2d, zero128):
    f = g.shape[1]
    parts = []
    for j in range(f // 128):
        gj = g[:, j * 128:(j + 1) * 128]
        p = _scat_kernel(gj, edges2d, zero128)
        parts.append((p[:N_NODES], p[NACC:NACC + N_NODES]))
    return parts


def _dinv_body(p0_ref, p1_ref, out_ref):
    out_ref[...] = lax.rsqrt(p0_ref[...] + p1_ref[...] + 1.0)


def _dinv(degp):
    p0 = degp[:N_NODES, :1]
    p1 = degp[NACC:NACC + N_NODES, :1]
    return pl.pallas_call(
        _dinv_body,
        grid=(N_NODES // ROW_BLK,),
        in_specs=[
            pl.BlockSpec((ROW_BLK, 1), lambda i: (i, 0)),
            pl.BlockSpec((ROW_BLK, 1), lambda i: (i, 0)),
        ],
        out_specs=pl.BlockSpec((ROW_BLK, 1), lambda i: (i, 0)),
        out_shape=jax.ShapeDtypeStruct((N_NODES, 1), jnp.float32),
    )(p0, p1)


def _mm_scale_body(x_ref, w_ref, dinv_ref, out_ref):
    h = jnp.dot(x_ref[...], w_ref[...], preferred_element_type=jnp.float32)
    out_ref[...] = h * dinv_ref[...]


def _mm_scale(x, w, dinv):
    n, fin = x.shape
    fout = w.shape[1]
    return pl.pallas_call(
        _mm_scale_body,
        grid=(n // ROW_BLK,),
        in_specs=[
            pl.BlockSpec((ROW_BLK, fin), lambda i: (i, 0)),
            pl.BlockSpec((fin, fout), lambda i: (0, 0)),
            pl.BlockSpec((ROW_BLK, 1), lambda i: (i, 0)),
        ],
        out_specs=pl.BlockSpec((ROW_BLK, fout), lambda i: (i, 0)),
        out_shape=jax.ShapeDtypeStruct((n, fout), jnp.float32),
    )(x, w, dinv)


def _relu_combine(part_refs, g_ref, dinv_ref, b_ref):
    nchunk = len(part_refs) // 2
    cols = []
    for j in range(nchunk):
        p0 = part_refs[2 * j][...]
        p1 = part_refs[2 * j + 1][...]
        gj = g_ref[:, j * 128:(j + 1) * 128]
        bj = b_ref[:, j * 128:(j + 1) * 128]
        cols.append((p0 + p1 + gj) * dinv_ref[...] + bj)
    x = cols[0] if nchunk == 1 else jnp.concatenate(cols, axis=1)
    return jnp.maximum(x, 0.0)


def _combine_mm(parts, g, dinv, b, w):
    n, fin = g.shape
    fout = w.shape[1]
    nchunk = len(parts)

    def body(*refs):
        part_refs = refs[:2 * nchunk]
        g_ref, dinv_ref, b_ref, w_ref, out_ref = refs[2 * nchunk:]
        x = _relu_combine(part_refs, g_ref, dinv_ref, b_ref)
        h = jnp.dot(x, w_ref[...], preferred_element_type=jnp.float32)
        out_ref[...] = h * dinv_ref[...]

    chunk_spec = pl.BlockSpec((ROW_BLK, 128), lambda i: (i, 0))
    in_specs = [chunk_spec] * (2 * nchunk) + [
        pl.BlockSpec((ROW_BLK, fin), lambda i: (i, 0)),
        pl.BlockSpec((ROW_BLK, 1), lambda i: (i, 0)),
        pl.BlockSpec((1, fin), lambda i: (0, 0)),
        pl.BlockSpec((fin, fout), lambda i: (0, 0)),
    ]
    flat = [p for pair in parts for p in pair]
    return pl.pallas_call(
        body,
        grid=(n // ROW_BLK,),
        in_specs=in_specs,
        out_specs=pl.BlockSpec((ROW_BLK, fout), lambda i: (i, 0)),
        out_shape=jax.ShapeDtypeStruct((n, fout), jnp.float32),
    )(*flat, g, dinv, b.reshape(1, fin), w)


def _combine_final(parts, g, dinv, b):
    n, f = g.shape
    nchunk = len(parts)

    def body(*refs):
        part_refs = refs[:2 * nchunk]
        g_ref, dinv_ref, b_ref, out_ref = refs[2 * nchunk:]
        out_ref[...] = _relu_combine(part_refs, g_ref, dinv_ref, b_ref)

    chunk_spec = pl.BlockSpec((ROW_BLK, 128), lambda i: (i, 0))
    in_specs = [chunk_spec] * (2 * nchunk) + [
        pl.BlockSpec((ROW_BLK, f), lambda i: (i, 0)),
        pl.BlockSpec((ROW_BLK, 1), lambda i: (i, 0)),
        pl.BlockSpec((1, f), lambda i: (0, 0)),
    ]
    flat = [p for pair in parts for p in pair]
    return pl.pallas_call(
        body,
        grid=(n // ROW_BLK,),
        in_specs=in_specs,
        out_specs=pl.BlockSpec((ROW_BLK, f), lambda i: (i, 0)),
        out_shape=jax.ShapeDtypeStruct((n, f), jnp.float32),
    )(*flat, g, dinv, b.reshape(1, f))


def kernel(v, edge_index, W1, b1, W2, b2, W3, b3):
    src2d = edge_index[0].reshape(NBLK_TOT, EB)
    dst2d = edge_index[1].reshape(NBLK_TOT, EB)
    edges2d = jnp.stack([src2d, dst2d], axis=1)
    zero128 = jnp.zeros((STRIPE, 128), jnp.float32)
    ones128 = jnp.ones((EB, 128), jnp.float32)

    degp = _deg_kernel(dst2d, ones128, zero128)
    dinv = _dinv(degp)

    g1 = _mm_scale(v, W1, dinv)
    g2 = _combine_mm(_sc_scatter(g1, edges2d, zero128), g1, dinv, b1, W2)
    g3 = _combine_mm(_sc_scatter(g2, edges2d, zero128), g2, dinv, b2, W3)
    return _combine_final(_sc_scatter(g3, edges2d, zero128), g3, dinv, b3)


# speedup vs baseline: 17.7888x; 1.1648x over previous
import functools
import jax
import jax.numpy as jnp
from jax import lax
from jax.experimental import pallas as pl
from jax.experimental.pallas import tpu as pltpu
from jax.experimental.pallas import tpu_sc as plsc

N_NODES = 10000
N_EDGES = 320000
ROW_BLK = 2000

NSC = 2
NSUB = 16
NW = NSC * NSUB
EB = 80
NT = 125
E_PAD = NW * NT * EB
NBLK_TOT = E_PAD // EB
NACC = 10240
STRIPE = NACC // NSUB

_mesh = plsc.VectorSubcoreMesh(core_axis_name="c", subcore_axis_name="s")


def _scat_body(g_hbm, edges_hbm, zero_hbm, out_hbm, *rest):
    idx2 = rest[0:4]
    rows = rest[4:8]
    acc = rest[8]
    si = rest[9:13]
    sg = rest[13:17]
    c = lax.axis_index("c")
    s = lax.axis_index("s")
    w = c * NSUB + s
    pltpu.sync_copy(zero_hbm, acc.at[pl.ds(s * STRIPE, STRIPE)])
    plsc.subcore_barrier()

    def fetch_idx(t, b):
        pltpu.async_copy(edges_hbm.at[w * NT + t], idx2[b], si[b])

    def issue_gather(t, b):
        pltpu.make_async_copy(edges_hbm.at[w * NT + t], idx2[b], si[b]).wait()
        pltpu.async_copy(g_hbm.at[idx2[b].at[0]], rows[b], sg[b])

    def consume(t, b):
        pltpu.make_async_copy(g_hbm.at[idx2[b].at[0]], rows[b], sg[b]).wait()
        pltpu.sync_copy(rows[b], acc.at[idx2[b].at[1]], add=True)

    for t in range(4):
        fetch_idx(t, t)
    for t in range(3):
        issue_gather(t, t)

    def body(i, carry):
        for bp in range(4):
            t = 4 * i + bp
            consume(t, bp)
            t4 = jnp.minimum(t + 4, NT - 1)
            pl.when(t + 4 < NT)(lambda: fetch_idx(t4, bp))
            t2 = jnp.minimum(t + 3, NT - 1)
            b2 = (bp + 3) % 4
            pl.when(t + 3 < NT)(lambda: issue_gather(t2, b2))
        return carry

    lax.fori_loop(0, (NT - 1) // 4, body, 0)
    consume(NT - 1, 0)

    plsc.subcore_barrier()
    row0 = c * NACC + s * STRIPE
    pltpu.sync_copy(acc.at[pl.ds(s * STRIPE, STRIPE)],
                    out_hbm.at[pl.ds(row0, STRIPE)])


_scat_kernel = pl.kernel(
    _scat_body,
    out_type=jax.ShapeDtypeStruct((NSC * NACC, 128), jnp.float32),
    mesh=_mesh,
    scratch_types=(
        [pltpu.VMEM((2, EB), jnp.int32)] * 4
        + [pltpu.VMEM((EB, 128), jnp.float32)] * 4
        + [pltpu.VMEM_SHARED((NACC, 128), jnp.float32)]
        + [pltpu.SemaphoreType.DMA] * 8
    ),
)


def _deg_body(dst_hbm, ones_hbm, zero_hbm, out_hbm, *rest):
    idxD = rest[0:4]
    ones_v = rest[4]
    acc = rest[5]
    si = rest[6:10]
    c = lax.axis_index("c")
    s = lax.axis_index("s")
    w = c * NSUB + s
    pltpu.sync_copy(ones_hbm, ones_v)
    pltpu.sync_copy(zero_hbm, acc.at[pl.ds(s * STRIPE, STRIPE)])
    plsc.subcore_barrier()

    def fetch_idx(t, b):
        pltpu.async_copy(dst_hbm.at[w * NT + t], idxD[b], si[b])

    def consume(t, b):
        pltpu.make_async_copy(dst_hbm.at[w * NT + t], idxD[b], si[b]).wait()
        pltpu.sync_copy(ones_v, acc.at[idxD[b]], add=True)

    for t in range(4):
        fetch_idx(t, t)

    def body(i, carry):
        for bp in range(4):
            t = 4 * i + bp
            consume(t, bp)
            t4 = jnp.minimum(t + 4, NT - 1)
            pl.when(t + 4 < NT)(lambda: fetch_idx(t4, bp))
        return carry

    lax.fori_loop(0, (NT - 1) // 4, body, 0)
    consume(NT - 1, 0)
    plsc.subcore_barrier()
    row0 = c * NACC + s * STRIPE
    pltpu.sync_copy(acc.at[pl.ds(s * STRIPE, STRIPE)],
                    out_hbm.at[pl.ds(row0, STRIPE)])


_deg_kernel = pl.kernel(
    _deg_body,
    out_type=jax.ShapeDtypeStruct((NSC * NACC, 128), jnp.float32),
    mesh=_mesh,
    scratch_types=(
        [pltpu.VMEM((EB,), jnp.int32)] * 4
        + [pltpu.VMEM((EB, 128), jnp.float32)]
        + [pltpu.VMEM_SHARED((NACC, 128), jnp.float32)]
        + [pltpu.SemaphoreType.DMA] * 4
    ),
)


def _sc_scatter(g, edges2d, zero128):
    f = g.shape[1]
    parts = []
    for j in range(f // 128):
        gj = g[:, j * 128:(j + 1) * 128]
        p = _scat_kernel(gj, edges2d, zero128)
        parts.append((p[:N_NODES], p[NACC:NACC + N_NODES]))
    return parts


def _dinv_body(p0_ref, p1_ref, out_ref):
    out_ref[...] = lax.rsqrt(p0_ref[...] + p1_ref[...] + 1.0)


def _dinv(degp):
    p0 = degp[:N_NODES, :1]
    p1 = degp[NACC:NACC + N_NODES, :1]
    return pl.pallas_call(
        _dinv_body,
        grid=(N_NODES // ROW_BLK,),
        in_specs=[
            pl.BlockSpec((ROW_BLK, 1), lambda i: (i, 0)),
            pl.BlockSpec((ROW_BLK, 1), lambda i: (i, 0)),
        ],
        out_specs=pl.BlockSpec((ROW_BLK, 1), lambda i: (i, 0)),
        out_shape=jax.ShapeDtypeStruct((N_NODES, 1), jnp.float32),
    )(p0, p1)


def _mm_scale_body(x_ref, w_ref, dinv_ref, out_ref):
    h = jnp.dot(x_ref[...], w_ref[...], preferred_element_type=jnp.float32)
    out_ref[...] = h * dinv_ref[...]


def _mm_scale(x, w, dinv):
    n, fin = x.shape
    fout = w.shape[1]
    return pl.pallas_call(
        _mm_scale_body,
        grid=(n // ROW_BLK,),
        in_specs=[
            pl.BlockSpec((ROW_BLK, fin), lambda i: (i, 0)),
            pl.BlockSpec((fin, fout), lambda i: (0, 0)),
            pl.BlockSpec((ROW_BLK, 1), lambda i: (i, 0)),
        ],
        out_specs=pl.BlockSpec((ROW_BLK, fout), lambda i: (i, 0)),
        out_shape=jax.ShapeDtypeStruct((n, fout), jnp.float32),
    )(x, w, dinv)


def _relu_combine(part_refs, g_ref, dinv_ref, b_ref):
    nchunk = len(part_refs) // 2
    cols = []
    for j in range(nchunk):
        p0 = part_refs[2 * j][...]
        p1 = part_refs[2 * j + 1][...]
        gj = g_ref[:, j * 128:(j + 1) * 128]
        bj = b_ref[:, j * 128:(j + 1) * 128]
        cols.append((p0 + p1 + gj) * dinv_ref[...] + bj)
    x = cols[0] if nchunk == 1 else jnp.concatenate(cols, axis=1)
    return jnp.maximum(x, 0.0)


def _combine_mm(parts, g, dinv, b, w):
    n, fin = g.shape
    fout = w.shape[1]
    nchunk = len(parts)

    def body(*refs):
        part_refs = refs[:2 * nchunk]
        g_ref, dinv_ref, b_ref, w_ref, out_ref = refs[2 * nchunk:]
        x = _relu_combine(part_refs, g_ref, dinv_ref, b_ref)
        h = jnp.dot(x, w_ref[...], preferred_element_type=jnp.float32)
        out_ref[...] = h * dinv_ref[...]

    chunk_spec = pl.BlockSpec((ROW_BLK, 128), lambda i: (i, 0))
    in_specs = [chunk_spec] * (2 * nchunk) + [
        pl.BlockSpec((ROW_BLK, fin), lambda i: (i, 0)),
        pl.BlockSpec((ROW_BLK, 1), lambda i: (i, 0)),
        pl.BlockSpec((1, fin), lambda i: (0, 0)),
        pl.BlockSpec((fin, fout), lambda i: (0, 0)),
    ]
    flat = [p for pair in parts for p in pair]
    return pl.pallas_call(
        body,
        grid=(n // ROW_BLK,),
        in_specs=in_specs,
        out_specs=pl.BlockSpec((ROW_BLK, fout), lambda i: (i, 0)),
        out_shape=jax.ShapeDtypeStruct((n, fout), jnp.float32),
    )(*flat, g, dinv, b.reshape(1, fin), w)


def _combine_final(parts, g, dinv, b):
    n, f = g.shape
    nchunk = len(parts)

    def body(*refs):
        part_refs = refs[:2 * nchunk]
        g_ref, dinv_ref, b_ref, out_ref = refs[2 * nchunk:]
        out_ref[...] = _relu_combine(part_refs, g_ref, dinv_ref, b_ref)

    chunk_spec = pl.BlockSpec((ROW_BLK, 128), lambda i: (i, 0))
    in_specs = [chunk_spec] * (2 * nchunk) + [
        pl.BlockSpec((ROW_BLK, f), lambda i: (i, 0)),
        pl.BlockSpec((ROW_BLK, 1), lambda i: (i, 0)),
        pl.BlockSpec((1, f), lambda i: (0, 0)),
    ]
    flat = [p for pair in parts for p in pair]
    return pl.pallas_call(
        body,
        grid=(n // ROW_BLK,),
        in_specs=in_specs,
        out_specs=pl.BlockSpec((ROW_BLK, f), lambda i: (i, 0)),
        out_shape=jax.ShapeDtypeStruct((n, f), jnp.float32),
    )(*flat, g, dinv, b.reshape(1, f))


def kernel(v, edge_index, W1, b1, W2, b2, W3, b3):
    src2d = edge_index[0].reshape(NBLK_TOT, EB)
    dst2d = edge_index[1].reshape(NBLK_TOT, EB)
    edges2d = jnp.stack([src2d, dst2d], axis=1)
    zero128 = jnp.zeros((STRIPE, 128), jnp.float32)
    ones128 = jnp.ones((EB, 128), jnp.float32)

    degp = _deg_kernel(dst2d, ones128, zero128)
    dinv = _dinv(degp)

    g1 = _mm_scale(v, W1, dinv)
    g2 = _combine_mm(_sc_scatter(g1, edges2d, zero128), g1, dinv, b1, W2)
    g3 = _combine_mm(_sc_scatter(g2, edges2d, zero128), g2, dinv, b2, W3)
    return _combine_final(_sc_scatter(g3, edges2d, zero128), g3, dinv, b3)


# speedup vs baseline: 17.7905x; 1.0001x over previous
import jax
import jax.numpy as jnp
from jax import lax
from jax.experimental import pallas as pl
from jax.experimental.pallas import tpu as pltpu
from jax.experimental.pallas import tpu_sc as plsc

N_NODES = 10000
N_EDGES = 320000
ROW_BLK = 2000

NSC = 2
NSUB = 16
NW = NSC * NSUB
EB = 80
NT = 125
E_PAD = NW * NT * EB
NBLK_TOT = E_PAD // EB
NACC = 10240
STRIPE = NACC // NSUB

_mesh = plsc.VectorSubcoreMesh(core_axis_name="c", subcore_axis_name="s")


def _scat_body(g_hbm, edges_hbm, zero_hbm, out_hbm, *rest):
    idx2 = rest[0:4]
    rows = rest[4:8]
    acc = rest[8]
    si = rest[9:13]
    sg = rest[13:17]
    c = lax.axis_index("c")
    s = lax.axis_index("s")
    w = c * NSUB + s
    pltpu.sync_copy(zero_hbm, acc.at[pl.ds(s * STRIPE, STRIPE)])
    plsc.subcore_barrier()

    def fetch_idx(t, b):
        pltpu.async_copy(edges_hbm.at[w * NT + t], idx2[b], si[b])

    def issue_gather(t, b):
        pltpu.make_async_copy(edges_hbm.at[w * NT + t], idx2[b], si[b]).wait()
        pltpu.async_copy(g_hbm.at[idx2[b].at[0]], rows[b], sg[b])

    def consume(t, b):
        pltpu.make_async_copy(g_hbm.at[idx2[b].at[0]], rows[b], sg[b]).wait()
        pltpu.sync_copy(rows[b], acc.at[idx2[b].at[1]], add=True)

    for t in range(4):
        fetch_idx(t, t)
    for t in range(3):
        issue_gather(t, t)

    def body(i, carry):
        for bp in range(4):
            t = 4 * i + bp
            consume(t, bp)
            t4 = jnp.minimum(t + 4, NT - 1)
            pl.when(t + 4 < NT)(lambda: fetch_idx(t4, bp))
            t2 = jnp.minimum(t + 3, NT - 1)
            b2 = (bp + 3) % 4
            pl.when(t + 3 < NT)(lambda: issue_gather(t2, b2))
        return carry

    lax.fori_loop(0, (NT - 1) // 4, body, 0)
    consume(NT - 1, 0)

    plsc.subcore_barrier()
    row0 = c * NACC + s * STRIPE
    pltpu.sync_copy(acc.at[pl.ds(s * STRIPE, STRIPE)],
                    out_hbm.at[pl.ds(row0, STRIPE)])


_scat_kernel = pl.kernel(
    _scat_body,
    out_type=jax.ShapeDtypeStruct((NSC * NACC, 128), jnp.float32),
    mesh=_mesh,
    scratch_types=(
        [pltpu.VMEM((2, EB), jnp.int32)] * 4
        + [pltpu.VMEM((EB, 128), jnp.float32)] * 4
        + [pltpu.VMEM_SHARED((NACC, 128), jnp.float32)]
        + [pltpu.SemaphoreType.DMA] * 8
    ),
)


def _deg_body(dst_hbm, ones_hbm, zero_hbm, out_hbm, *rest):
    idxD = rest[0:4]
    ones_v = rest[4]
    acc = rest[5]
    si = rest[6:10]
    c = lax.axis_index("c")
    s = lax.axis_index("s")
    w = c * NSUB + s
    pltpu.sync_copy(ones_hbm, ones_v)
    pltpu.sync_copy(zero_hbm, acc.at[pl.ds(s * STRIPE, STRIPE)])
    plsc.subcore_barrier()

    def fetch_idx(t, b):
        pltpu.async_copy(dst_hbm.at[w * NT + t], idxD[b], si[b])

    def consume(t, b):
        pltpu.make_async_copy(dst_hbm.at[w * NT + t], idxD[b], si[b]).wait()
        pltpu.sync_copy(ones_v, acc.at[idxD[b]], add=True)

    for t in range(4):
        fetch_idx(t, t)

    def body(i, carry):
        for bp in range(4):
            t = 4 * i + bp
            consume(t, bp)
            t4 = jnp.minimum(t + 4, NT - 1)
            pl.when(t + 4 < NT)(lambda: fetch_idx(t4, bp))
        return carry

    lax.fori_loop(0, (NT - 1) // 4, body, 0)
    consume(NT - 1, 0)
    plsc.subcore_barrier()
    row0 = c * NACC + s * STRIPE
    pltpu.sync_copy(acc.at[pl.ds(s * STRIPE, STRIPE)],
                    out_hbm.at[pl.ds(row0, STRIPE)])


_deg_kernel = pl.kernel(
    _deg_body,
    out_type=jax.ShapeDtypeStruct((NSC * NACC, 128), jnp.float32),
    mesh=_mesh,
    scratch_types=(
        [pltpu.VMEM((EB,), jnp.int32)] * 4
        + [pltpu.VMEM((EB, 128), jnp.float32)]
        + [pltpu.VMEM_SHARED((NACC, 128), jnp.float32)]
        + [pltpu.SemaphoreType.DMA] * 4
    ),
)


def _sc_scatter(g, edges2d, zero128):
    f = g.shape[1]
    parts = []
    for j in range(f // 128):
        gj = g[:, j * 128:(j + 1) * 128]
        p = _scat_kernel(gj, edges2d, zero128)
        parts.append((p[:N_NODES], p[NACC:NACC + N_NODES]))
    return parts


def _dinv_body(p0_ref, p1_ref, out_ref):
    out_ref[...] = lax.rsqrt(p0_ref[...] + p1_ref[...] + 1.0)


def _dinv(degp):
    p0 = degp[:N_NODES, :1]
    p1 = degp[NACC:NACC + N_NODES, :1]
    return pl.pallas_call(
        _dinv_body,
        grid=(N_NODES // ROW_BLK,),
        in_specs=[
            pl.BlockSpec((ROW_BLK, 1), lambda i: (i, 0)),
            pl.BlockSpec((ROW_BLK, 1), lambda i: (i, 0)),
        ],
        out_specs=pl.BlockSpec((ROW_BLK, 1), lambda i: (i, 0)),
        out_shape=jax.ShapeDtypeStruct((N_NODES, 1), jnp.float32),
    )(p0, p1)


def _mm_scale_body(x_ref, w_ref, dinv_ref, out_ref):
    h = jnp.dot(x_ref[...], w_ref[...], preferred_element_type=jnp.float32)
    out_ref[...] = h * dinv_ref[...]


def _mm_scale(x, w, dinv):
    n, fin = x.shape
    fout = w.shape[1]
    return pl.pallas_call(
        _mm_scale_body,
        grid=(n // ROW_BLK,),
        in_specs=[
            pl.BlockSpec((ROW_BLK, fin), lambda i: (i, 0)),
            pl.BlockSpec((fin, fout), lambda i: (0, 0)),
            pl.BlockSpec((ROW_BLK, 1), lambda i: (i, 0)),
        ],
        out_specs=pl.BlockSpec((ROW_BLK, fout), lambda i: (i, 0)),
        out_shape=jax.ShapeDtypeStruct((n, fout), jnp.float32),
    )(x, w, dinv)


def _relu_combine(part_refs, g_ref, dinv_ref, b_ref):
    nchunk = len(part_refs) // 2
    cols = []
    for j in range(nchunk):
        p0 = part_refs[2 * j][...]
        p1 = part_refs[2 * j + 1][...]
        gj = g_ref[:, j * 128:(j + 1) * 128]
        bj = b_ref[:, j * 128:(j + 1) * 128]
        cols.append((p0 + p1 + gj) * dinv_ref[...] + bj)
    x = cols[0] if nchunk == 1 else jnp.concatenate(cols, axis=1)
    return jnp.maximum(x, 0.0)


def _combine_mm(parts, g, dinv, b, w):
    n, fin = g.shape
    fout = w.shape[1]
    nchunk = len(parts)

    def body(*refs):
        part_refs = refs[:2 * nchunk]
        g_ref, dinv_ref, b_ref, w_ref, out_ref = refs[2 * nchunk:]
        x = _relu_combine(part_refs, g_ref, dinv_ref, b_ref)
        h = jnp.dot(x, w_ref[...], preferred_element_type=jnp.float32)
        out_ref[...] = h * dinv_ref[...]

    chunk_spec = pl.BlockSpec((ROW_BLK, 128), lambda i: (i, 0))
    in_specs = [chunk_spec] * (2 * nchunk) + [
        pl.BlockSpec((ROW_BLK, fin), lambda i: (i, 0)),
        pl.BlockSpec((ROW_BLK, 1), lambda i: (i, 0)),
        pl.BlockSpec((1, fin), lambda i: (0, 0)),
        pl.BlockSpec((fin, fout), lambda i: (0, 0)),
    ]
    flat = [p for pair in parts for p in pair]
    return pl.pallas_call(
        body,
        grid=(n // ROW_BLK,),
        in_specs=in_specs,
        out_specs=pl.BlockSpec((ROW_BLK, fout), lambda i: (i, 0)),
        out_shape=jax.ShapeDtypeStruct((n, fout), jnp.float32),
    )(*flat, g, dinv, b.reshape(1, fin), w)


def _combine_final(parts, g, dinv, b):
    n, f = g.shape
    nchunk = len(parts)

    def body(*refs):
        part_refs = refs[:2 * nchunk]
        g_ref, dinv_ref, b_ref, out_ref = refs[2 * nchunk:]
        out_ref[...] = _relu_combine(part_refs, g_ref, dinv_ref, b_ref)

    chunk_spec = pl.BlockSpec((ROW_BLK, 128), lambda i: (i, 0))
    in_specs = [chunk_spec] * (2 * nchunk) + [
        pl.BlockSpec((ROW_BLK, f), lambda i: (i, 0)),
        pl.BlockSpec((ROW_BLK, 1), lambda i: (i, 0)),
        pl.BlockSpec((1, f), lambda i: (0, 0)),
    ]
    flat = [p for pair in parts for p in pair]
    return pl.pallas_call(
        body,
        grid=(n // ROW_BLK,),
        in_specs=in_specs,
        out_specs=pl.BlockSpec((ROW_BLK, f), lambda i: (i, 0)),
        out_shape=jax.ShapeDtypeStruct((n, f), jnp.float32),
    )(*flat, g, dinv, b.reshape(1, f))


def kernel(v, edge_index, W1, b1, W2, b2, W3, b3):
    src2d = edge_index[0].reshape(NBLK_TOT, EB)
    dst2d = edge_index[1].reshape(NBLK_TOT, EB)
    edges2d = jnp.stack([src2d, dst2d], axis=1)
    zero128 = jnp.zeros((STRIPE, 128), jnp.float32)
    ones128 = jnp.ones((EB, 128), jnp.float32)

    degp = _deg_kernel(dst2d, ones128, zero128)
    dinv = _dinv(degp)

    g1 = _mm_scale(v, W1, dinv)
    g2 = _combine_mm(_sc_scatter(g1, edges2d, zero128), g1, dinv, b1, W2)
    g3 = _combine_mm(_sc_scatter(g2, edges2d, zero128), g2, dinv, b2, W3)
    return _combine_final(_sc_scatter(g3, edges2d, zero128), g3, dinv, b3)
